# baseline (speedup 1.0000x reference)
import jax
import jax.numpy as jnp
from jax.experimental import pallas as pl


def _gate_kernel(x_ref, wg1_ref, bg1_ref, wg2_ref, bg2_ref, mask_ref):
    x = x_ref[...].astype(jnp.bfloat16)
    h = jnp.dot(x, wg1_ref[...].astype(jnp.bfloat16),
                preferred_element_type=jnp.float32)
    h = jnp.maximum(h + bg1_ref[...], 0.0).astype(jnp.bfloat16)
    logits = jnp.dot(h, wg2_ref[...].astype(jnp.bfloat16),
                     preferred_element_type=jnp.float32)
    logits = logits + bg2_ref[...]
    m = jnp.max(logits, axis=-1, keepdims=True)
    ex = jnp.exp(logits - m)
    att = ex / jnp.sum(ex, axis=-1, keepdims=True)
    E = att.shape[-1]
    idx = jax.lax.broadcasted_iota(jnp.int32, att.shape, 1)
    m1 = jnp.max(att, axis=-1, keepdims=True)
    i1 = jnp.min(jnp.where(att == m1, idx, E), axis=-1, keepdims=True)
    att2 = jnp.where(idx == i1, -jnp.inf, att)
    m2 = jnp.max(att2, axis=-1, keepdims=True)
    i2 = jnp.min(jnp.where(att2 == m2, idx, E), axis=-1, keepdims=True)
    z = jnp.exp(m2 - m1)
    w1 = 1.0 / (1.0 + z)
    w2 = z / (1.0 + z)
    mask = jnp.where(idx == i1, w1, 0.0) + jnp.where(idx == i2, w2, 0.0)
    mask_ref[...] = mask


def _expert_kernel(x_ref, mask_ref, we1_ref, be1_ref, we2_ref, be2_ref,
                   out_ref):
    x = x_ref[...]
    E = we1_ref.shape[0]
    acc = jnp.zeros(out_ref.shape[1:], jnp.float32)
    for e in range(E):
        h = jnp.dot(x, we1_ref[e], preferred_element_type=jnp.float32)
        h = jnp.maximum(h + be1_ref[e][None, :], 0.0).astype(jnp.bfloat16)
        p = jnp.dot(h, we2_ref[e], preferred_element_type=jnp.float32)
        p = p + be2_ref[e][None, :]
        w = mask_ref[:, e][:, None]
        acc = acc + (w * p).T
    out_ref[0] = acc


def kernel(hidden_state, Wg1, bg1, Wg2, bg2, We1, be1, We2, be2):
    B, N, D = hidden_state.shape
    E = Wg2.shape[-1]
    F = We1.shape[-1]
    T = We2.shape[-1]
    M = B * N
    TM = 512
    n_tiles = M // TM
    tiles_per_b = N // TM

    x = hidden_state.reshape(M, D)

    mask = pl.pallas_call(
        _gate_kernel,
        grid=(n_tiles,),
        in_specs=[
            pl.BlockSpec((TM, D), lambda j: (j, 0)),
            pl.BlockSpec((D, F), lambda j: (0, 0)),
            pl.BlockSpec((1, F), lambda j: (0, 0)),
            pl.BlockSpec((F, E), lambda j: (0, 0)),
            pl.BlockSpec((1, E), lambda j: (0, 0)),
        ],
        out_specs=pl.BlockSpec((TM, E), lambda j: (j, 0)),
        out_shape=jax.ShapeDtypeStruct((M, E), jnp.float32),
    )(x, Wg1, bg1[None, :], Wg2, bg2[None, :])

    x_bf = x.astype(jnp.bfloat16)
    we1_bf = We1.astype(jnp.bfloat16)
    we2_bf = We2.astype(jnp.bfloat16)

    final_t = pl.pallas_call(
        _expert_kernel,
        grid=(n_tiles,),
        in_specs=[
            pl.BlockSpec((TM, D), lambda j: (j, 0)),
            pl.BlockSpec((TM, E), lambda j: (j, 0)),
            pl.BlockSpec((E, D, F), lambda j: (0, 0, 0)),
            pl.BlockSpec((E, F), lambda j: (0, 0)),
            pl.BlockSpec((E, F, T), lambda j: (0, 0, 0)),
            pl.BlockSpec((E, T), lambda j: (0, 0)),
        ],
        out_specs=pl.BlockSpec(
            (1, T, TM), lambda j: (j // tiles_per_b, 0, j % tiles_per_b)),
        out_shape=jax.ShapeDtypeStruct((B, T, N), jnp.float32),
    )(x_bf, mask, we1_bf, be1, we2_bf, be2)

    final_pred = final_t[..., None]
    att_experts = mask.reshape(B, N, E, 1)
    return final_pred, att_experts


# speedup vs baseline: 1.0425x; 1.0425x over previous
import jax
import jax.numpy as jnp
from jax.experimental import pallas as pl


def _gate_kernel(x_ref, wg1_ref, bg1_ref, wg2_ref, bg2_ref, mask_ref,
                 xbf_ref):
    x = x_ref[...].astype(jnp.bfloat16)
    xbf_ref[...] = x
    h = jnp.dot(x, wg1_ref[...].astype(jnp.bfloat16),
                preferred_element_type=jnp.float32)
    h = jnp.maximum(h + bg1_ref[...], 0.0).astype(jnp.bfloat16)
    logits = jnp.dot(h, wg2_ref[...].astype(jnp.bfloat16),
                     preferred_element_type=jnp.float32)
    logits = logits + bg2_ref[...]
    m = jnp.max(logits, axis=-1, keepdims=True)
    ex = jnp.exp(logits - m)
    att = ex / jnp.sum(ex, axis=-1, keepdims=True)
    E = att.shape[-1]
    idx = jax.lax.broadcasted_iota(jnp.int32, att.shape, 1)
    m1 = jnp.max(att, axis=-1, keepdims=True)
    i1 = jnp.min(jnp.where(att == m1, idx, E), axis=-1, keepdims=True)
    att2 = jnp.where(idx == i1, -jnp.inf, att)
    m2 = jnp.max(att2, axis=-1, keepdims=True)
    i2 = jnp.min(jnp.where(att2 == m2, idx, E), axis=-1, keepdims=True)
    z = jnp.exp(m2 - m1)
    w1 = 1.0 / (1.0 + z)
    w2 = z / (1.0 + z)
    mask = jnp.where(idx == i1, w1, 0.0) + jnp.where(idx == i2, w2, 0.0)
    mask_ref[...] = mask


def _expert_kernel(x_ref, mask_ref, we1_ref, be1_ref, we2_ref, be2_ref,
                   out_ref):
    x = x_ref[...]
    E = we1_ref.shape[0]
    acc = jnp.zeros(out_ref.shape[1:], jnp.float32)
    for e in range(E):
        h = jnp.dot(x, we1_ref[e], preferred_element_type=jnp.float32)
        h = jnp.maximum(h + be1_ref[e][None, :], 0.0).astype(jnp.bfloat16)
        p = jnp.dot(h, we2_ref[e], preferred_element_type=jnp.float32)
        p = p + be2_ref[e][None, :]
        w = mask_ref[:, e][:, None]
        acc = acc + (w * p).T
    out_ref[0] = acc


def kernel(hidden_state, Wg1, bg1, Wg2, bg2, We1, be1, We2, be2):
    B, N, D = hidden_state.shape
    E = Wg2.shape[-1]
    F = We1.shape[-1]
    T = We2.shape[-1]
    M = B * N
    TM = 512
    n_tiles = M // TM
    tiles_per_b = N // TM

    x = hidden_state.reshape(M, D)

    mask, x_bf = pl.pallas_call(
        _gate_kernel,
        grid=(n_tiles,),
        in_specs=[
            pl.BlockSpec((TM, D), lambda j: (j, 0)),
            pl.BlockSpec((D, F), lambda j: (0, 0)),
            pl.BlockSpec((1, F), lambda j: (0, 0)),
            pl.BlockSpec((F, E), lambda j: (0, 0)),
            pl.BlockSpec((1, E), lambda j: (0, 0)),
        ],
        out_specs=[
            pl.BlockSpec((TM, E), lambda j: (j, 0)),
            pl.BlockSpec((TM, D), lambda j: (j, 0)),
        ],
        out_shape=[
            jax.ShapeDtypeStruct((M, E), jnp.float32),
            jax.ShapeDtypeStruct((M, D), jnp.bfloat16),
        ],
    )(x, Wg1, bg1[None, :], Wg2, bg2[None, :])

    we1_bf = We1.astype(jnp.bfloat16)
    we2_bf = We2.astype(jnp.bfloat16)

    final_t = pl.pallas_call(
        _expert_kernel,
        grid=(n_tiles,),
        in_specs=[
            pl.BlockSpec((TM, D), lambda j: (j, 0)),
            pl.BlockSpec((TM, E), lambda j: (j, 0)),
            pl.BlockSpec((E, D, F), lambda j: (0, 0, 0)),
            pl.BlockSpec((E, F), lambda j: (0, 0)),
            pl.BlockSpec((E, F, T), lambda j: (0, 0, 0)),
            pl.BlockSpec((E, T), lambda j: (0, 0)),
        ],
        out_specs=pl.BlockSpec(
            (1, T, TM), lambda j: (j // tiles_per_b, 0, j % tiles_per_b)),
        out_shape=jax.ShapeDtypeStruct((B, T, N), jnp.float32),
    )(x_bf, mask, we1_bf, be1, we2_bf, be2)

    final_pred = final_t[..., None]
    att_experts = mask.reshape(B, N, E, 1)
    return final_pred, att_experts
